# baseline (speedup 1.0000x reference)
import functools

import jax
import jax.numpy as jnp
from jax import lax
from jax.experimental import pallas as pl
from jax.experimental.pallas import tpu as pltpu
from jax.experimental.pallas import tpu_sc as plsc

N_NODES = 10000
N_EDGES = 320000
D_IN, D_HID, D_OUT = 128, 128, 64

NC, NS, L = 2, 16, 16
NW = NC * NS
K = 128
C = 84
CH_REAL = N_EDGES // K
RPS = 632
N_PAD = NS * RPS

ROW_BLK = 1000
GRID = N_NODES // ROW_BLK


def _mm_body(x_ref, w_ref, b_ref, o_ref):
    o_ref[...] = (
        jnp.dot(x_ref[...], w_ref[...], preferred_element_type=jnp.float32)
        + b_ref[...]
    )


def _mm_split_body(x_ref, w_ref, b_ref, o_ref):
    o_ref[0] = (
        jnp.dot(x_ref[...], w_ref[0], preferred_element_type=jnp.float32)
        + b_ref[0]
    )


def _mm_split(x, w, b):
    dout = w.shape[1]
    dh = dout // 2
    din = x.shape[1]
    ws = jnp.stack([w[:, :dh], w[:, dh:]])
    bs = b.reshape(2, 1, dh)
    return pl.pallas_call(
        _mm_split_body,
        grid=(GRID, 2),
        in_specs=[
            pl.BlockSpec((ROW_BLK, din), lambda i, j: (i, 0)),
            pl.BlockSpec((1, din, dh), lambda i, j: (j, 0, 0)),
            pl.BlockSpec((1, 1, dh), lambda i, j: (j, 0, 0)),
        ],
        out_specs=pl.BlockSpec((1, ROW_BLK, dh), lambda i, j: (j, i, 0)),
        out_shape=jax.ShapeDtypeStruct((2, N_PAD, dh), jnp.float32),
    )(x, ws, bs)


def _mm_fused_body(p_ref, w_ref, b_ref, o_ref):
    dh = p_ref.shape[2]
    h_lo = jnp.maximum(p_ref[0], 0.0)
    h_hi = jnp.maximum(p_ref[1], 0.0)
    o_ref[...] = (
        jnp.dot(h_lo, w_ref[:dh], preferred_element_type=jnp.float32)
        + jnp.dot(h_hi, w_ref[dh:], preferred_element_type=jnp.float32)
        + b_ref[...]
    )


def _mm_fused(p, w, b):
    d = p.shape[2]
    dout = w.shape[1]
    return pl.pallas_call(
        _mm_fused_body,
        grid=(GRID,),
        in_specs=[
            pl.BlockSpec((2, ROW_BLK, d), lambda i: (0, i, 0)),
            pl.BlockSpec(w.shape, lambda i: (0, 0)),
            pl.BlockSpec((1, dout), lambda i: (0, 0)),
        ],
        out_specs=pl.BlockSpec((ROW_BLK, dout), lambda i: (i, 0)),
        out_shape=jax.ShapeDtypeStruct((N_PAD, dout), jnp.float32),
    )(p, w, b.reshape(1, dout))


def _pair_add_body(p_ref, o_ref):
    o_ref[...] = p_ref[0] + p_ref[1]


def _pair_add(p):
    d = p.shape[2]
    return pl.pallas_call(
        _pair_add_body,
        grid=(GRID,),
        in_specs=[pl.BlockSpec((2, ROW_BLK, d), lambda i: (0, i, 0))],
        out_specs=pl.BlockSpec((ROW_BLK, d), lambda i: (i, 0)),
        out_shape=jax.ShapeDtypeStruct((N_NODES, d), jnp.float32),
    )(p)


def _make_agg(D, dsplit):
    mesh = plsc.VectorSubcoreMesh(core_axis_name="c", subcore_axis_name="s")
    TCH = (NW * C) // NS if dsplit else C
    T = TCH // 6

    @functools.partial(
        pl.kernel,
        out_type=jax.ShapeDtypeStruct((NC * N_PAD, D), jnp.float32),
        mesh=mesh,
        scratch_types=[
            [pltpu.VMEM((2, K), jnp.int32) for _ in range(6)],
            [pltpu.VMEM((K,), jnp.float32) for _ in range(6)],
            [pltpu.VMEM((K, D), jnp.float32) for _ in range(3)],
            [pltpu.VMEM((K, D), jnp.float32) for _ in range(2)],
            pltpu.VMEM_SHARED((N_PAD, D), jnp.float32),
            pltpu.VMEM_SHARED((N_PAD, D), jnp.float32),
            [pltpu.SemaphoreType.DMA for _ in range(6)],
            [pltpu.SemaphoreType.DMA for _ in range(3)],
            [pltpu.SemaphoreType.DMA for _ in range(2)],
        ],
        compiler_params=pltpu.CompilerParams(
            use_tc_tiling_on_sc=False, needs_layout_passes=False
        ),
    )
    def agg(sup, eidx, ew, zeros, out,
            idx_bufs, ew_bufs, rows_bufs, scaled_bufs, acc, sup_sp,
            idx_sems, gather_sems, scatter_sems):
        cid = lax.axis_index("c")
        sid = lax.axis_index("s")
        wid = sid * NC + cid
        row0 = sid * RPS
        pltpu.sync_copy(zeros.at[pl.ds(row0, RPS)], acc.at[pl.ds(row0, RPS)])
        sup_base = cid * N_PAD + row0 if dsplit else row0
        pltpu.sync_copy(sup.at[pl.ds(sup_base, RPS)], sup_sp.at[pl.ds(row0, RPS)])
        plsc.subcore_barrier()
        g0 = sid * TCH if dsplit else wid * C

        def live(c):
            return g0 + c < CH_REAL

        def issue_idx(c, slot):
            pltpu.async_copy(
                eidx.at[:, pl.ds((g0 + c) * K, K)], idx_bufs[slot],
                idx_sems[slot],
            )
            pltpu.async_copy(
                ew.at[pl.ds((g0 + c) * K, K)], ew_bufs[slot], idx_sems[slot]
            )

        def wait_idx(slot):
            pltpu.make_async_copy(
                eidx.at[:, pl.ds(0, K)], idx_bufs[slot], idx_sems[slot]
            ).wait()
            pltpu.make_async_copy(
                ew.at[pl.ds(0, K)], ew_bufs[slot], idx_sems[slot]
            ).wait()

        def issue_gather(slot6, rslot):
            pltpu.async_copy(
                sup_sp.at[idx_bufs[slot6].at[0]], rows_bufs[rslot],
                gather_sems[rslot],
            )

        def wait_gather(rslot):
            pltpu.make_async_copy(
                sup.at[pl.ds(0, K)], rows_bufs[rslot], gather_sems[rslot]
            ).wait()

        def issue_scatter(slot6, sslot):
            pltpu.async_copy(
                scaled_bufs[sslot], acc.at[idx_bufs[slot6].at[1]],
                scatter_sems[sslot], add=True,
            )

        def wait_scatter(sslot):
            pltpu.make_async_copy(
                sup.at[pl.ds(0, K)], scaled_bufs[sslot], scatter_sems[sslot]
            ).wait()

        def scale(rows_v, out_v, ew_v):
            @plsc.parallel_loop(0, K // L)
            def body(g):
                ewg = ew_v[pl.ds(g * L, L)]
                for l in range(L):
                    w = ewg[l]
                    row = g * L + l
                    for j in range(D // L):
                        sl = pl.ds(j * L, L)
                        out_v[row, sl] = rows_v[row, sl] * w

        for x in range(4):
            @pl.when(live(x))
            def _(x=x):
                issue_idx(x, x)
        for x in range(2):
            @pl.when(live(x))
            def _(x=x):
                wait_idx(x)
                issue_gather(x, x)

        def block(c, q):
            @pl.when((c + 2 < TCH) & live(c + 2))
            def _():
                wait_idx((q + 2) % 6)
                issue_gather((q + 2) % 6, (q + 2) % 3)

            @pl.when((c >= 2) & live(c - 2))
            def _():
                wait_scatter(q % 2)

            @pl.when((c + 4 < TCH) & live(c + 4))
            def _():
                issue_idx(c + 4, (q + 4) % 6)

            @pl.when(live(c))
            def _():
                wait_gather(q % 3)
                scale(rows_bufs[q % 3], scaled_bufs[q % 2], ew_bufs[q])
                issue_scatter(q, q % 2)

        def step(t, carry):
            c = 6 * t
            for q in range(6):
                block(c + q, q)
            return carry

        lax.fori_loop(0, T, step, 0)
        for x in (TCH - 2, TCH - 1):
            @pl.when(live(x))
            def _(x=x):
                wait_scatter(x % 2)

        plsc.subcore_barrier()
        pltpu.sync_copy(
            acc.at[pl.ds(row0, RPS)],
            out.at[pl.ds(cid * N_PAD + row0, RPS)],
        )

    return agg


_agg_hid = _make_agg(D_HID // 2, dsplit=True)
_agg_out = _make_agg(D_OUT, dsplit=False)


def kernel(fea, edge_index, edge_weight, W1, b1, W2, b2):
    z64 = jnp.zeros((N_PAD, 64), jnp.float32)

    sup1 = _mm_split(fea, W1, b1).reshape(NC * N_PAD, D_HID // 2)
    h_halves = _agg_hid(sup1, edge_index, edge_weight, z64)
    sup2 = _mm_fused(h_halves.reshape(NC, N_PAD, D_HID // 2), W2, b2)
    p2 = _agg_out(sup2, edge_index, edge_weight, z64)
    return _pair_add(p2.reshape(NC, N_PAD, D_OUT))


# speedup vs baseline: 1.1076x; 1.1076x over previous
import functools

import jax
import jax.numpy as jnp
from jax import lax
from jax.experimental import pallas as pl
from jax.experimental.pallas import tpu as pltpu
from jax.experimental.pallas import tpu_sc as plsc

N_NODES = 10000
N_EDGES = 320000
D_IN, D_HID, D_OUT = 128, 128, 64

NC, NS, L = 2, 16, 16
NW = NC * NS
K = 128
C = 80
E_PAD = NW * K * C
RPS = 632
N_PAD = NS * RPS

ROW_BLK = 1000
GRID = N_NODES // ROW_BLK


def _mm_body(x_ref, w_ref, b_ref, o_ref):
    o_ref[...] = (
        jnp.dot(x_ref[...], w_ref[...], preferred_element_type=jnp.float32)
        + b_ref[...]
    )


def _mm_split_body(x_ref, w_ref, b_ref, o_ref):
    o_ref[0] = (
        jnp.dot(x_ref[...], w_ref[0], preferred_element_type=jnp.float32)
        + b_ref[0]
    )


def _mm_split(x, w, b):
    dout = w.shape[1]
    dh = dout // 2
    din = x.shape[1]
    ws = jnp.stack([w[:, :dh], w[:, dh:]])
    bs = b.reshape(2, 1, dh)
    return pl.pallas_call(
        _mm_split_body,
        grid=(GRID, 2),
        in_specs=[
            pl.BlockSpec((ROW_BLK, din), lambda i, j: (i, 0)),
            pl.BlockSpec((1, din, dh), lambda i, j: (j, 0, 0)),
            pl.BlockSpec((1, 1, dh), lambda i, j: (j, 0, 0)),
        ],
        out_specs=pl.BlockSpec((1, ROW_BLK, dh), lambda i, j: (j, i, 0)),
        out_shape=jax.ShapeDtypeStruct((2, N_PAD, dh), jnp.float32),
    )(x, ws, bs)


def _mm_fused_body(p_ref, w_ref, b_ref, o_ref):
    dh = p_ref.shape[2]
    h_lo = jnp.maximum(p_ref[0], 0.0)
    h_hi = jnp.maximum(p_ref[1], 0.0)
    o_ref[...] = (
        jnp.dot(h_lo, w_ref[:dh], preferred_element_type=jnp.float32)
        + jnp.dot(h_hi, w_ref[dh:], preferred_element_type=jnp.float32)
        + b_ref[...]
    )


def _mm_fused(p, w, b):
    d = p.shape[2]
    dout = w.shape[1]
    return pl.pallas_call(
        _mm_fused_body,
        grid=(GRID,),
        in_specs=[
            pl.BlockSpec((2, ROW_BLK, d), lambda i: (0, i, 0)),
            pl.BlockSpec(w.shape, lambda i: (0, 0)),
            pl.BlockSpec((1, dout), lambda i: (0, 0)),
        ],
        out_specs=pl.BlockSpec((ROW_BLK, dout), lambda i: (i, 0)),
        out_shape=jax.ShapeDtypeStruct((N_PAD, dout), jnp.float32),
    )(p, w, b.reshape(1, dout))


def _pair_add_body(p_ref, o_ref):
    o_ref[...] = p_ref[0] + p_ref[1]


def _pair_add(p):
    d = p.shape[2]
    return pl.pallas_call(
        _pair_add_body,
        grid=(GRID,),
        in_specs=[pl.BlockSpec((2, ROW_BLK, d), lambda i: (0, i, 0))],
        out_specs=pl.BlockSpec((ROW_BLK, d), lambda i: (i, 0)),
        out_shape=jax.ShapeDtypeStruct((N_NODES, d), jnp.float32),
    )(p)


def _make_agg(D, dsplit):
    mesh = plsc.VectorSubcoreMesh(core_axis_name="c", subcore_axis_name="s")
    TCH = (NW * C) // NS if dsplit else C
    T = TCH // 4

    @functools.partial(
        pl.kernel,
        out_type=jax.ShapeDtypeStruct((NC * N_PAD, D), jnp.float32),
        mesh=mesh,
        scratch_types=[
            [pltpu.VMEM((2, K), jnp.int32) for _ in range(4)],
            [pltpu.VMEM((K,), jnp.float32) for _ in range(4)],
            [pltpu.VMEM((K, D), jnp.float32) for _ in range(2)],
            [pltpu.VMEM((K, D), jnp.float32) for _ in range(2)],
            pltpu.VMEM_SHARED((N_PAD, D), jnp.float32),
            pltpu.VMEM_SHARED((N_PAD, D), jnp.float32),
            [pltpu.SemaphoreType.DMA for _ in range(4)],
            [pltpu.SemaphoreType.DMA for _ in range(2)],
            [pltpu.SemaphoreType.DMA for _ in range(2)],
        ],
        compiler_params=pltpu.CompilerParams(
            use_tc_tiling_on_sc=False, needs_layout_passes=False
        ),
    )
    def agg(sup, eidx, ew, zeros, out,
            idx_bufs, ew_bufs, rows_bufs, scaled_bufs, acc, sup_sp,
            idx_sems, gather_sems, scatter_sems):
        cid = lax.axis_index("c")
        sid = lax.axis_index("s")
        wid = sid * NC + cid
        row0 = sid * RPS
        pltpu.sync_copy(zeros.at[pl.ds(row0, RPS)], acc.at[pl.ds(row0, RPS)])
        sup_base = cid * N_PAD + row0 if dsplit else row0
        pltpu.sync_copy(sup.at[pl.ds(sup_base, RPS)], sup_sp.at[pl.ds(row0, RPS)])
        plsc.subcore_barrier()
        g0 = sid * TCH if dsplit else wid * C

        def issue_idx(c, slot):
            pltpu.async_copy(
                eidx.at[:, pl.ds((g0 + c) * K, K)], idx_bufs[slot],
                idx_sems[slot],
            )
            pltpu.async_copy(
                ew.at[pl.ds((g0 + c) * K, K)], ew_bufs[slot], idx_sems[slot]
            )

        def wait_idx(slot):
            pltpu.make_async_copy(
                eidx.at[:, pl.ds(0, K)], idx_bufs[slot], idx_sems[slot]
            ).wait()
            pltpu.make_async_copy(
                ew.at[pl.ds(0, K)], ew_bufs[slot], idx_sems[slot]
            ).wait()

        def issue_gather(slot4, rslot):
            pltpu.async_copy(
                sup_sp.at[idx_bufs[slot4].at[0]], rows_bufs[rslot],
                gather_sems[rslot],
            )

        def wait_gather(rslot):
            pltpu.make_async_copy(
                sup.at[pl.ds(0, K)], rows_bufs[rslot], gather_sems[rslot]
            ).wait()

        def issue_scatter(slot4, sslot):
            pltpu.async_copy(
                scaled_bufs[sslot], acc.at[idx_bufs[slot4].at[1]],
                scatter_sems[sslot], add=True,
            )

        def wait_scatter(sslot):
            pltpu.make_async_copy(
                sup.at[pl.ds(0, K)], scaled_bufs[sslot], scatter_sems[sslot]
            ).wait()

        def scale(rows_v, out_v, ew_v):
            @plsc.parallel_loop(0, K // L)
            def body(g):
                ewg = ew_v[pl.ds(g * L, L)]
                for l in range(L):
                    w = ewg[l]
                    row = g * L + l
                    for j in range(D // L):
                        sl = pl.ds(j * L, L)
                        out_v[row, sl] = rows_v[row, sl] * w

        issue_idx(0, 0)
        issue_idx(1, 1)
        wait_idx(0)
        issue_gather(0, 0)

        def block(c, q):
            X = q % 2

            @pl.when(c + 1 < TCH)
            def _():
                wait_idx((q + 1) % 4)
                issue_gather((q + 1) % 4, (q + 1) % 2)

            @pl.when(c >= 2)
            def _():
                wait_scatter(X)

            @pl.when(c + 2 < TCH)
            def _():
                issue_idx(c + 2, (q + 2) % 4)

            wait_gather(X)
            scale(rows_bufs[X], scaled_bufs[X], ew_bufs[q])
            issue_scatter(q, X)

        def step(t, carry):
            c = 4 * t
            for q in range(4):
                block(c + q, q)
            return carry

        lax.fori_loop(0, T, step, 0)
        wait_scatter(0)
        wait_scatter(1)

        plsc.subcore_barrier()
        pltpu.sync_copy(
            acc.at[pl.ds(row0, RPS)],
            out.at[pl.ds(cid * N_PAD + row0, RPS)],
        )

    return agg


_agg_hid = _make_agg(D_HID // 2, dsplit=True)
_agg_out = _make_agg(D_OUT, dsplit=False)


def kernel(fea, edge_index, edge_weight, W1, b1, W2, b2):
    pad = E_PAD - N_EDGES
    eidx = jnp.concatenate(
        [edge_index, jnp.zeros((2, pad), jnp.int32)], axis=1
    )
    ew = jnp.concatenate([edge_weight, jnp.zeros((pad,), jnp.float32)])
    z64 = jnp.zeros((N_PAD, 64), jnp.float32)

    sup1 = _mm_split(fea, W1, b1).reshape(NC * N_PAD, D_HID // 2)
    h_halves = _agg_hid(sup1, eidx, ew, z64)
    sup2 = _mm_fused(h_halves.reshape(NC, N_PAD, D_HID // 2), W2, b2)
    p2 = _agg_out(sup2, eidx, ew, z64)
    return _pair_add(p2.reshape(NC, N_PAD, D_OUT))
